# baseline (speedup 1.0000x reference)
import functools

import jax
import jax.numpy as jnp
from jax import lax
from jax.experimental import pallas as pl
from jax.experimental.pallas import tpu as pltpu
from jax.experimental.pallas import tpu_sc as plsc

DIM = 64
NB = 16384
NJ = 20
NC, NS = 2, 16
NW = NC * NS
BBLK = 128
NBB = NB // BBLK
NTC = NJ * NBB
TCW = NTC // NW


def _make_sc_gather():
    mesh = plsc.VectorSubcoreMesh(
        core_axis_name="c", subcore_axis_name="s", num_cores=NC, num_subcores=NS
    )

    @functools.partial(
        pl.kernel,
        out_type=jax.ShapeDtypeStruct((NJ, 8, NBB, 8, BBLK), jnp.float32),
        mesh=mesh,
        scratch_types=[
            pltpu.VMEM((2, BBLK), jnp.int32),
            pltpu.VMEM((2, BBLK, DIM), jnp.float32),
            pltpu.VMEM((2, 8, 8, BBLK + 8), jnp.float32),
            pltpu.SemaphoreType.DMA((2,)),
            pltpu.SemaphoreType.DMA((2,)),
            pltpu.SemaphoreType.DMA((2,)),
        ],
        compiler_params=pltpu.CompilerParams(
            use_tc_tiling_on_sc=False, needs_layout_passes=False
        ),
    )
    def gather_kernel(table_hbm, idxt_hbm, out_hbm, idx_v, gbuf, tbuf, sem_i, sem_g, sem_w):
        wid = lax.axis_index("s") * NC + lax.axis_index("c")
        tc0 = wid * TCW

        lane = lax.iota(jnp.int32, 16)
        fvecs = [lane + (16 * fg) for fg in range(4)]
        ffvs = [lax.shift_right_logical(v, 3) for v in fvecs]
        fmvs = [lax.bitwise_and(v, 7) for v in fvecs]

        def idx_src(t):
            tc = tc0 + t
            j = tc // NBB
            bb = tc % NBB
            return idxt_hbm.at[j, pl.ds(bb * BBLK, BBLK)]

        pltpu.async_copy(idx_src(0), idx_v.at[0], sem_i.at[0])
        pltpu.async_copy(idx_src(1), idx_v.at[1], sem_i.at[1])
        pltpu.make_async_copy(idx_src(0), idx_v.at[0], sem_i.at[0]).wait()
        pltpu.async_copy(table_hbm.at[idx_v.at[0]], gbuf.at[0], sem_g.at[0])

        @pl.loop(0, TCW, step=2)
        def _(t2):
            for par in range(2):
                t = t2 + par
                nxt = 1 - par
                tc = tc0 + t
                j = tc // NBB
                bb = tc % NBB

                pltpu.make_async_copy(
                    table_hbm.at[idx_v.at[par]], gbuf.at[par], sem_g.at[par]
                ).wait()

                @pl.when(t + 2 < TCW)
                def _():
                    pltpu.async_copy(idx_src(t + 2), idx_v.at[par], sem_i.at[par])

                @pl.when(t >= 2)
                def _():
                    pltpu.make_async_copy(
                        tbuf.at[par, :, :, pl.ds(0, BBLK)],
                        out_hbm.at[j, :, bb],
                        sem_w.at[par],
                    ).wait()

                @pl.loop(0, BBLK, unroll=8)
                def _(c):
                    cv = jnp.full((16,), c, jnp.int32)
                    for fg in range(4):
                        vals = gbuf[par, c, pl.ds(fg * 16, 16)]
                        plsc.store_scatter(
                            tbuf.at[par], [ffvs[fg], fmvs[fg], cv], vals
                        )

                pltpu.async_copy(
                    tbuf.at[par, :, :, pl.ds(0, BBLK)],
                    out_hbm.at[j, :, bb],
                    sem_w.at[par],
                )

                @pl.when(t + 1 < TCW)
                def _():
                    pltpu.make_async_copy(
                        idx_src(t + 1), idx_v.at[nxt], sem_i.at[nxt]
                    ).wait()
                    pltpu.async_copy(
                        table_hbm.at[idx_v.at[nxt]], gbuf.at[nxt], sem_g.at[nxt]
                    )

        for par in range(2):
            t = TCW - 2 + par
            tc = tc0 + t
            pltpu.make_async_copy(
                tbuf.at[par, :, :, pl.ds(0, BBLK)],
                out_hbm.at[tc // NBB, :, tc % NBB],
                sem_w.at[par],
            ).wait()

    return gather_kernel


_sc_gather = _make_sc_gather()


@jax.jit
def kernel(class_number, embedding):
    idx_t = jnp.transpose(class_number, (1, 0)).astype(jnp.int32)
    y = _sc_gather(embedding, idx_t)
    return y.transpose(2, 4, 0, 1, 3).reshape(NB, NJ, DIM)


# speedup vs baseline: 1.4094x; 1.4094x over previous
import functools

import jax
import jax.numpy as jnp
from jax import lax
from jax.experimental import pallas as pl
from jax.experimental.pallas import tpu as pltpu
from jax.experimental.pallas import tpu_sc as plsc

DIM = 64
NB = 16384
NJ = 20
NC, NS = 2, 16
NW = NC * NS
BBLK = 128
NBB = NB // BBLK
NTC = NJ * NBB
TCW = NTC // NW


def _make_sc_gather():
    mesh = plsc.VectorSubcoreMesh(
        core_axis_name="c", subcore_axis_name="s", num_cores=NC, num_subcores=NS
    )

    @functools.partial(
        pl.kernel,
        out_type=jax.ShapeDtypeStruct((NJ, 8, NBB, 8, BBLK), jnp.float32),
        mesh=mesh,
        scratch_types=[
            pltpu.VMEM((2, BBLK), jnp.int32),
            pltpu.VMEM((2, BBLK, DIM), jnp.float32),
            pltpu.VMEM((2, 8, 8, BBLK + 8), jnp.float32),
            pltpu.SemaphoreType.DMA((2,)),
            pltpu.SemaphoreType.DMA((2,)),
            pltpu.SemaphoreType.DMA((2,)),
        ],
        compiler_params=pltpu.CompilerParams(
            use_tc_tiling_on_sc=False, needs_layout_passes=False
        ),
    )
    def gather_kernel(table_hbm, idxt_hbm, out_hbm, idx_v, gbuf, tbuf, sem_i, sem_g, sem_w):
        wid = lax.axis_index("s") * NC + lax.axis_index("c")
        tc0 = wid * TCW

        lane = lax.iota(jnp.int32, 16)
        fvecs = [lane + (16 * fg) for fg in range(4)]
        ffvs = [lax.shift_right_logical(v, 3) for v in fvecs]
        fmvs = [lax.bitwise_and(v, 7) for v in fvecs]

        def idx_src(t):
            tc = tc0 + t
            j = tc // NBB
            bb = tc % NBB
            return idxt_hbm.at[j, pl.ds(bb * BBLK, BBLK)]

        pltpu.async_copy(idx_src(0), idx_v.at[0], sem_i.at[0])
        pltpu.async_copy(idx_src(1), idx_v.at[1], sem_i.at[1])
        pltpu.make_async_copy(idx_src(0), idx_v.at[0], sem_i.at[0]).wait()
        pltpu.async_copy(table_hbm.at[idx_v.at[0]], gbuf.at[0], sem_g.at[0])

        @pl.loop(0, TCW, step=2)
        def _(t2):
            for par in range(2):
                t = t2 + par
                nxt = 1 - par
                tc = tc0 + t
                j = tc // NBB
                bb = tc % NBB

                pltpu.make_async_copy(
                    table_hbm.at[idx_v.at[par]], gbuf.at[par], sem_g.at[par]
                ).wait()

                @pl.when(t + 1 < TCW)
                def _():
                    pltpu.make_async_copy(
                        idx_src(t + 1), idx_v.at[nxt], sem_i.at[nxt]
                    ).wait()
                    pltpu.async_copy(
                        table_hbm.at[idx_v.at[nxt]], gbuf.at[nxt], sem_g.at[nxt]
                    )

                @pl.when(t + 2 < TCW)
                def _():
                    pltpu.async_copy(idx_src(t + 2), idx_v.at[par], sem_i.at[par])

                @pl.when(t >= 2)
                def _():
                    pltpu.make_async_copy(
                        tbuf.at[par, :, :, pl.ds(0, BBLK)],
                        out_hbm.at[j, :, bb],
                        sem_w.at[par],
                    ).wait()

                @pl.loop(0, BBLK, unroll=8)
                def _(c):
                    cv = jnp.full((16,), c, jnp.int32)
                    for fg in range(4):
                        vals = gbuf[par, c, pl.ds(fg * 16, 16)]
                        plsc.store_scatter(
                            tbuf.at[par], [ffvs[fg], fmvs[fg], cv], vals
                        )

                pltpu.async_copy(
                    tbuf.at[par, :, :, pl.ds(0, BBLK)],
                    out_hbm.at[j, :, bb],
                    sem_w.at[par],
                )

        for par in range(2):
            t = TCW - 2 + par
            tc = tc0 + t
            pltpu.make_async_copy(
                tbuf.at[par, :, :, pl.ds(0, BBLK)],
                out_hbm.at[tc // NBB, :, tc % NBB],
                sem_w.at[par],
            ).wait()

    return gather_kernel


_sc_gather = _make_sc_gather()


@jax.jit
def kernel(class_number, embedding):
    idx_t = jnp.transpose(class_number, (1, 0)).astype(jnp.int32)
    y = _sc_gather(embedding, idx_t)
    return y.transpose(2, 4, 0, 1, 3).reshape(NB, NJ, DIM)
